# baseline (speedup 1.0000x reference)
import functools

import jax
import jax.numpy as jnp
from jax import lax
from jax.experimental import pallas as pl
from jax.experimental.pallas import tpu as pltpu
from jax.experimental.pallas import tpu_sc as plsc

N_VOCAB = 1000000
N_EMB = 64
NC = 2
NS = 16
NW = NC * NS
CHUNK = 128


def _gather_body(idx_hbm, emb_hbm, out_hbm, idx_v, rows_v, gsem):
    k = idx_hbm.shape[1]
    wid = lax.axis_index("s") * NC + lax.axis_index("c")
    pltpu.sync_copy(idx_hbm.at[wid], idx_v)

    @pl.loop(0, k)
    def _(j):
        pltpu.async_copy(emb_hbm.at[idx_v.at[j]], rows_v, gsem).wait()
        pltpu.sync_copy(rows_v, out_hbm.at[wid, j])


def kernel(xs, emb):
    b, s = xs.shape
    n = b * s
    assert n % (NW * CHUNK) == 0
    k = n // (NW * CHUNK)
    idx = xs.reshape(NW, k, CHUNK)

    mesh = plsc.VectorSubcoreMesh(core_axis_name="c", subcore_axis_name="s")
    run = functools.partial(
        pl.kernel,
        out_type=jax.ShapeDtypeStruct((NW, k, CHUNK, N_EMB), jnp.float32),
        mesh=mesh,
        scratch_types=[
            pltpu.VMEM((k, CHUNK), jnp.int32),
            pltpu.VMEM((CHUNK, N_EMB), jnp.float32),
            pltpu.SemaphoreType.DMA,
        ],
        compiler_params=pltpu.CompilerParams(use_tc_tiling_on_sc=False),
    )(_gather_body)
    out = run(idx, emb)
    return out.reshape(b, s, N_EMB)


# speedup vs baseline: 1.1167x; 1.1167x over previous
import functools

import jax
import jax.numpy as jnp
from jax import lax
from jax.experimental import pallas as pl
from jax.experimental.pallas import tpu as pltpu
from jax.experimental.pallas import tpu_sc as plsc

N_VOCAB = 1000000
N_EMB = 64
NC = 2
NS = 16
NW = NC * NS
CHUNK = 128


NBUF = 8


def _gather_body(idx_hbm, emb_hbm, out_hbm, idx_v, rows_v, gsem):
    k = idx_hbm.shape[1]
    wid = lax.axis_index("s") * NC + lax.axis_index("c")
    pltpu.sync_copy(idx_hbm.at[wid], idx_v)

    for b in range(NBUF):
        pltpu.async_copy(emb_hbm.at[idx_v.at[b]], rows_v.at[b], gsem.at[b])

    @pl.loop(0, k, step=NBUF)
    def _(j):
        for b in range(NBUF):
            g = j + b
            pltpu.make_async_copy(
                emb_hbm.at[idx_v.at[b]], rows_v.at[b], gsem.at[b]
            ).wait()
            pltpu.sync_copy(rows_v.at[b], out_hbm.at[wid, g])

            @pl.when(g + NBUF < k)
            def _():
                pltpu.async_copy(
                    emb_hbm.at[idx_v.at[g + NBUF]], rows_v.at[b], gsem.at[b]
                )


def kernel(xs, emb):
    b, s = xs.shape
    n = b * s
    assert n % (NW * CHUNK) == 0
    k = n // (NW * CHUNK)
    idx = xs.reshape(NW, k, CHUNK)

    mesh = plsc.VectorSubcoreMesh(core_axis_name="c", subcore_axis_name="s")
    run = functools.partial(
        pl.kernel,
        out_type=jax.ShapeDtypeStruct((NW, k, CHUNK, N_EMB), jnp.float32),
        mesh=mesh,
        scratch_types=[
            pltpu.VMEM((k, CHUNK), jnp.int32),
            pltpu.VMEM((NBUF, CHUNK, N_EMB), jnp.float32),
            pltpu.SemaphoreType.DMA((NBUF,)),
        ],
        compiler_params=pltpu.CompilerParams(use_tc_tiling_on_sc=False),
    )(_gather_body)
    out = run(idx, emb)
    return out.reshape(b, s, N_EMB)
